# baseline (speedup 1.0000x reference)
import functools

import jax
import jax.numpy as jnp
from jax import lax
from jax.experimental import pallas as pl
from jax.experimental.pallas import tpu as pltpu
from jax.experimental.pallas import tpu_sc as plsc

_BATCH = 16384
_EMB = 32
_NC = 2
_NS = 16
_NW = _NC * _NS
_BPW = _BATCH // _NW
_CH = 64
_L = 16

_BT = 2048


def _sc_gather(user_idx, item_idx, ut_t, it_t):
    mesh = plsc.VectorSubcoreMesh(core_axis_name="c", subcore_axis_name="s")

    @functools.partial(
        pl.kernel,
        out_type=(
            jax.ShapeDtypeStruct((_BATCH // 4, 128), jnp.float32),
            jax.ShapeDtypeStruct((_BATCH // 4, 128), jnp.float32),
        ),
        mesh=mesh,
        compiler_params=pltpu.CompilerParams(use_tc_tiling_on_sc=False,
                                             needs_layout_passes=False),
        scratch_types=[
            pltpu.VMEM((_BPW,), jnp.int32),
            pltpu.VMEM((_BPW,), jnp.int32),
            pltpu.VMEM((_EMB, _CH * _L), jnp.float32),
            pltpu.VMEM((_EMB, _CH * _L), jnp.float32),
            pltpu.VMEM((_BPW // 4, 128), jnp.float32),
            pltpu.VMEM((_BPW // 4, 128), jnp.float32),
            pltpu.SemaphoreType.DMA,
            pltpu.SemaphoreType.DMA,
        ],
    )
    def gather(uidx_hbm, iidx_hbm, utab_hbm, itab_hbm, uout_hbm, iout_hbm,
               uidx_v, iidx_v, uslab_v, islab_v,
               uout_v, iout_v, usem, isem):
        wid = lax.axis_index("s") * _NC + lax.axis_index("c")
        base = pl.multiple_of(wid * _BPW, _BPW)
        base4 = pl.multiple_of(wid * (_BPW // 4), _BPW // 4)
        pltpu.sync_copy(uidx_hbm.at[pl.ds(base, _BPW)], uidx_v)
        pltpu.sync_copy(iidx_hbm.at[pl.ds(base, _BPW)], iidx_v)

        def fire(g, c):
            off = pl.multiple_of(c * _CH + g * _L, _L)
            uvec = uidx_v[pl.ds(off, _L)]
            ivec = iidx_v[pl.ds(off, _L)]
            for k2 in range(_L):
                uoff = pl.multiple_of((uvec[k2] // _L) * _L, _L)
                ioff = pl.multiple_of((ivec[k2] // _L) * _L, _L)
                dst = pl.ds(pl.multiple_of((g * _L + k2) * _L, _L), _L)
                pltpu.async_copy(utab_hbm.at[:, pl.ds(uoff, _L)],
                                 uslab_v.at[:, dst], usem)
                pltpu.async_copy(itab_hbm.at[:, pl.ds(ioff, _L)],
                                 islab_v.at[:, dst], isem)
            return c

        def extract(slab_v, idx_v, out_v, c):
            for g in range(_CH // _L):
                lrow = lax.iota(jnp.int32, _L) + g * _L
                off = pl.multiple_of(c * _CH + g * _L, _L)
                lane = lax.bitwise_and(idx_v[pl.ds(off, _L)], _L - 1)
                cols = lrow * _L + lane
                grow = lrow + c * _CH
                drow = lax.shift_right_logical(grow, 2)
                dcol0 = lax.bitwise_and(grow, 3) * _EMB
                for j in range(_EMB):
                    vals = plsc.load_gather(
                        slab_v, [jnp.full((_L,), j, jnp.int32), cols])
                    plsc.store_scatter(out_v, [drow, dcol0 + j], vals)

        def chunk(c, _):
            lax.fori_loop(0, _CH // _L, fire, c)
            pltpu.make_async_copy(
                utab_hbm.at[:, pl.ds(0, _CH * _L)], uslab_v, usem).wait()
            extract(uslab_v, uidx_v, uout_v, c)
            pltpu.make_async_copy(
                itab_hbm.at[:, pl.ds(0, _CH * _L)], islab_v, isem).wait()
            extract(islab_v, iidx_v, iout_v, c)
            return 0

        lax.fori_loop(0, _BPW // _CH, chunk, 0)

        pltpu.sync_copy(uout_v, uout_hbm.at[pl.ds(base4, _BPW // 4)])
        pltpu.sync_copy(iout_v, iout_hbm.at[pl.ds(base4, _BPW // 4)])

    return gather(user_idx, item_idx, ut_t, it_t)


def _mlp_body(u_ref, i_ref, w1u_ref, w1i_ref, b1_ref, w2_ref, b2_ref,
              w3_ref, b3_ref, o_ref):
    dn = (((1,), (1,)), ((), ()))
    x1 = lax.dot_general(u_ref[...], w1u_ref[...], dn,
                         preferred_element_type=jnp.float32)
    x1 = x1 + lax.dot_general(i_ref[...], w1i_ref[...], dn,
                              preferred_element_type=jnp.float32)
    x1 = jnp.maximum(x1 + b1_ref[...], 0.0)
    x2 = lax.dot_general(x1, w2_ref[...], dn,
                         preferred_element_type=jnp.float32)
    x2 = jnp.maximum(x2 + b2_ref[...], 0.0)
    z = jnp.sum(x2 * w3_ref[...], axis=1, keepdims=True)
    z = z + b3_ref[0]
    o_ref[...] = 1.0 / (1.0 + jnp.exp(-z))


def _tc_mlp(u_emb, i_emb, W1u, W1i, b1r, W2, b2r, W3, b3):
    grid = (_BATCH // _BT,)
    full = lambda shape: pl.BlockSpec(shape, lambda i: (0, 0))
    return pl.pallas_call(
        _mlp_body,
        grid=grid,
        in_specs=[
            pl.BlockSpec((_BT, _EMB), lambda i: (i, 0)),
            pl.BlockSpec((_BT, _EMB), lambda i: (i, 0)),
            full(W1u.shape),
            full(W1i.shape),
            full(b1r.shape),
            full(W2.shape),
            full(b2r.shape),
            full(W3.shape),
            pl.BlockSpec(memory_space=pltpu.SMEM),
        ],
        out_specs=pl.BlockSpec((_BT, 1), lambda i: (i, 0)),
        out_shape=jax.ShapeDtypeStruct((_BATCH, 1), jnp.float32),
    )(u_emb, i_emb, W1u, W1i, b1r, W2, b2r, W3, b3)


def kernel(user_idx, item_idx, user_table, item_table, W1, b1, W2, b2, W3, b3):
    uidx = user_idx.astype(jnp.int32)
    iidx = item_idx.astype(jnp.int32)
    u_out, i_out = _sc_gather(uidx, iidx, user_table.T, item_table.T)
    u_emb = u_out.reshape(_BATCH, _EMB)
    i_emb = i_out.reshape(_BATCH, _EMB)
    W1u = W1[:, :_EMB]
    W1i = W1[:, _EMB:]
    return _tc_mlp(u_emb, i_emb, W1u, W1i,
                   b1.reshape(1, -1), W2, b2.reshape(1, -1),
                   W3, b3)


# speedup vs baseline: 4.8919x; 4.8919x over previous
import functools

import jax
import jax.numpy as jnp
from jax import lax
from jax.experimental import pallas as pl
from jax.experimental.pallas import tpu as pltpu
from jax.experimental.pallas import tpu_sc as plsc

_BATCH = 16384
_EMB = 32
_NC = 2
_NS = 16
_NW = _NC * _NS
_BPW = _BATCH // _NW

_BT = 2048


def _sc_gather(user_idx, item_idx, user_table, item_table):
    mesh = plsc.VectorSubcoreMesh(core_axis_name="c", subcore_axis_name="s")

    @functools.partial(
        pl.kernel,
        out_type=(
            jax.ShapeDtypeStruct((_BATCH, _EMB), jnp.bfloat16),
            jax.ShapeDtypeStruct((_BATCH, _EMB), jnp.bfloat16),
        ),
        mesh=mesh,
        compiler_params=pltpu.CompilerParams(use_tc_tiling_on_sc=False),
        scratch_types=[
            pltpu.VMEM((_BPW,), jnp.int32),
            pltpu.VMEM((_BPW,), jnp.int32),
            pltpu.VMEM((_BPW, _EMB), jnp.bfloat16),
            pltpu.VMEM((_BPW, _EMB), jnp.bfloat16),
            pltpu.SemaphoreType.DMA,
            pltpu.SemaphoreType.DMA,
        ],
    )
    def gather(uidx_hbm, iidx_hbm, utab_hbm, itab_hbm, uout_hbm, iout_hbm,
               uidx_v, iidx_v, urows_v, irows_v, usem, isem):
        wid = lax.axis_index("s") * _NC + lax.axis_index("c")
        base = pl.multiple_of(wid * _BPW, _BPW)
        pltpu.sync_copy(uidx_hbm.at[pl.ds(base, _BPW)], uidx_v)
        pltpu.sync_copy(iidx_hbm.at[pl.ds(base, _BPW)], iidx_v)
        cu = pltpu.async_copy(utab_hbm.at[uidx_v], urows_v, usem)
        ci = pltpu.async_copy(itab_hbm.at[iidx_v], irows_v, isem)
        cu.wait()
        ci.wait()
        pltpu.sync_copy(urows_v, uout_hbm.at[pl.ds(base, _BPW)])
        pltpu.sync_copy(irows_v, iout_hbm.at[pl.ds(base, _BPW)])

    return gather(user_idx, item_idx, user_table, item_table)


def _mlp_body(u_ref, i_ref, w1u_ref, w1i_ref, b1_ref, w2_ref, b2_ref,
              w3_ref, b3_ref, o_ref):
    dn = (((1,), (1,)), ((), ()))
    u = u_ref[...].astype(jnp.float32)
    it = i_ref[...].astype(jnp.float32)
    x1 = lax.dot_general(u, w1u_ref[...], dn,
                         preferred_element_type=jnp.float32)
    x1 = x1 + lax.dot_general(it, w1i_ref[...], dn,
                              preferred_element_type=jnp.float32)
    x1 = jnp.maximum(x1 + b1_ref[...], 0.0)
    x2 = lax.dot_general(x1, w2_ref[...], dn,
                         preferred_element_type=jnp.float32)
    x2 = jnp.maximum(x2 + b2_ref[...], 0.0)
    z = jnp.sum(x2 * w3_ref[...], axis=1, keepdims=True)
    z = z + b3_ref[0]
    o_ref[...] = 1.0 / (1.0 + jnp.exp(-z))


def _tc_mlp(u_emb, i_emb, W1u, W1i, b1r, W2, b2r, W3, b3):
    grid = (_BATCH // _BT,)
    full = lambda shape: pl.BlockSpec(shape, lambda i: (0, 0))
    return pl.pallas_call(
        _mlp_body,
        grid=grid,
        in_specs=[
            pl.BlockSpec((_BT, _EMB), lambda i: (i, 0)),
            pl.BlockSpec((_BT, _EMB), lambda i: (i, 0)),
            full(W1u.shape),
            full(W1i.shape),
            full(b1r.shape),
            full(W2.shape),
            full(b2r.shape),
            full(W3.shape),
            pl.BlockSpec(memory_space=pltpu.SMEM),
        ],
        out_specs=pl.BlockSpec((_BT, 1), lambda i: (i, 0)),
        out_shape=jax.ShapeDtypeStruct((_BATCH, 1), jnp.float32),
    )(u_emb, i_emb, W1u, W1i, b1r, W2, b2r, W3, b3)


def kernel(user_idx, item_idx, user_table, item_table, W1, b1, W2, b2, W3, b3):
    uidx = user_idx.astype(jnp.int32)
    iidx = item_idx.astype(jnp.int32)
    ut16 = user_table.astype(jnp.bfloat16)
    it16 = item_table.astype(jnp.bfloat16)
    u_emb, i_emb = _sc_gather(uidx, iidx, ut16, it16)
    W1u = W1[:, :_EMB]
    W1i = W1[:, _EMB:]
    return _tc_mlp(u_emb, i_emb, W1u, W1i,
                   b1.reshape(1, -1), W2, b2.reshape(1, -1),
                   W3, b3)


# speedup vs baseline: 7.3761x; 1.5078x over previous
import functools

import jax
import jax.numpy as jnp
from jax import lax
from jax.experimental import pallas as pl
from jax.experimental.pallas import tpu as pltpu
from jax.experimental.pallas import tpu_sc as plsc

_BATCH = 16384
_EMB = 32
_NC = 2
_NS = 16
_NW = _NC * _NS
_BPW = _BATCH // _NW
_CH = 32
_L = 16

_BT = 2048


def _sc_gather(user_idx, item_idx, user_table, item_table, dummy):
    mesh = plsc.VectorSubcoreMesh(core_axis_name="c", subcore_axis_name="s")

    @functools.partial(
        pl.kernel,
        out_type=(
            jax.ShapeDtypeStruct((_BATCH // 4, 128), jnp.float32),
            jax.ShapeDtypeStruct((_BATCH // 4, 128), jnp.float32),
        ),
        mesh=mesh,
        compiler_params=pltpu.CompilerParams(use_tc_tiling_on_sc=True,
                                             needs_layout_passes=False),
        scratch_types=[
            pltpu.VMEM((_BPW,), jnp.int32),
            pltpu.VMEM((_BPW,), jnp.int32),
            pltpu.VMEM((_CH, 8, _EMB), jnp.float32),
            pltpu.VMEM((_CH, 8, _EMB), jnp.float32),
            pltpu.VMEM((_BPW // 4, 128), jnp.float32),
            pltpu.VMEM((_BPW // 4, 128), jnp.float32),
            pltpu.SemaphoreType.DMA,
            pltpu.SemaphoreType.DMA,
        ],
    )
    def gather(uidx_hbm, iidx_hbm, utab_hbm, itab_hbm, dummy_hbm,
               uout_hbm, iout_hbm,
               uidx_v, iidx_v, uslab_v, islab_v,
               uout_v, iout_v, usem, isem):
        wid = lax.axis_index("s") * _NC + lax.axis_index("c")
        base = pl.multiple_of(wid * _BPW, _BPW)
        base4 = pl.multiple_of(wid * (_BPW // 4), _BPW // 4)
        pltpu.sync_copy(uidx_hbm.at[pl.ds(base, _BPW)], uidx_v)
        pltpu.sync_copy(iidx_hbm.at[pl.ds(base, _BPW)], iidx_v)

        def fire(g, c):
            off = pl.multiple_of(c * _CH + g * _L, _L)
            uvec = uidx_v[pl.ds(off, _L)]
            ivec = iidx_v[pl.ds(off, _L)]
            for k2 in range(_L):
                urow = pl.multiple_of((uvec[k2] // 8) * 8, 8)
                irow = pl.multiple_of((ivec[k2] // 8) * 8, 8)
                pltpu.async_copy(utab_hbm.at[pl.ds(urow, 8)],
                                 uslab_v.at[g * _L + k2], usem)
                pltpu.async_copy(itab_hbm.at[pl.ds(irow, 8)],
                                 islab_v.at[g * _L + k2], isem)
            return c

        def extract(slab_v, idx_v, out_v, c):
            for g in range(_CH // _L):
                lrow = lax.iota(jnp.int32, _L) + g * _L
                off = pl.multiple_of(c * _CH + g * _L, _L)
                sub = lax.bitwise_and(idx_v[pl.ds(off, _L)], 7)
                grow = lrow + c * _CH
                drow = lax.shift_right_logical(grow, 2)
                dcol0 = lax.bitwise_and(grow, 3) * _EMB
                for j in range(_EMB):
                    vals = plsc.load_gather(
                        slab_v, [lrow, sub, jnp.full((_L,), j, jnp.int32)])
                    plsc.store_scatter(out_v, [drow, dcol0 + j], vals)

        def chunk(c, _):
            lax.fori_loop(0, _CH // _L, fire, c)
            pltpu.make_async_copy(dummy_hbm, uslab_v, usem).wait()
            extract(uslab_v, uidx_v, uout_v, c)
            pltpu.make_async_copy(dummy_hbm, islab_v, isem).wait()
            extract(islab_v, iidx_v, iout_v, c)
            return 0

        lax.fori_loop(0, _BPW // _CH, chunk, 0)

        pltpu.sync_copy(uout_v, uout_hbm.at[pl.ds(base4, _BPW // 4)])
        pltpu.sync_copy(iout_v, iout_hbm.at[pl.ds(base4, _BPW // 4)])

    return gather(user_idx, item_idx, user_table, item_table, dummy)


def _mlp_body(u_ref, i_ref, w1u_ref, w1i_ref, b1_ref, w2_ref, b2_ref,
              w3_ref, b3_ref, o_ref):
    dn = (((1,), (1,)), ((), ()))
    x1 = lax.dot_general(u_ref[...], w1u_ref[...], dn,
                         preferred_element_type=jnp.float32)
    x1 = x1 + lax.dot_general(i_ref[...], w1i_ref[...], dn,
                              preferred_element_type=jnp.float32)
    x1 = jnp.maximum(x1 + b1_ref[...], 0.0)
    x2 = lax.dot_general(x1, w2_ref[...], dn,
                         preferred_element_type=jnp.float32)
    x2 = jnp.maximum(x2 + b2_ref[...], 0.0)
    z = jnp.sum(x2 * w3_ref[...], axis=1, keepdims=True)
    z = z + b3_ref[0]
    o_ref[...] = 1.0 / (1.0 + jnp.exp(-z))


def _tc_mlp(u_emb, i_emb, W1u, W1i, b1r, W2, b2r, W3, b3):
    grid = (_BATCH // _BT,)
    full = lambda shape: pl.BlockSpec(shape, lambda i: (0, 0))
    return pl.pallas_call(
        _mlp_body,
        grid=grid,
        in_specs=[
            pl.BlockSpec((_BT, _EMB), lambda i: (i, 0)),
            pl.BlockSpec((_BT, _EMB), lambda i: (i, 0)),
            full(W1u.shape),
            full(W1i.shape),
            full(b1r.shape),
            full(W2.shape),
            full(b2r.shape),
            full(W3.shape),
            pl.BlockSpec(memory_space=pltpu.SMEM),
        ],
        out_specs=pl.BlockSpec((_BT, 1), lambda i: (i, 0)),
        out_shape=jax.ShapeDtypeStruct((_BATCH, 1), jnp.float32),
    )(u_emb, i_emb, W1u, W1i, b1r, W2, b2r, W3, b3)


def kernel(user_idx, item_idx, user_table, item_table, W1, b1, W2, b2, W3, b3):
    uidx = user_idx.astype(jnp.int32)
    iidx = item_idx.astype(jnp.int32)
    dummy = jnp.zeros((_CH, 8, _EMB), jnp.float32)
    u_out, i_out = _sc_gather(uidx, iidx, user_table, item_table, dummy)
    u_emb = u_out.reshape(_BATCH, _EMB)
    i_emb = i_out.reshape(_BATCH, _EMB)
    W1u = W1[:, :_EMB]
    W1i = W1[:, _EMB:]
    return _tc_mlp(u_emb, i_emb, W1u, W1i,
                   b1.reshape(1, -1), W2, b2.reshape(1, -1),
                   W3, b3)


# speedup vs baseline: 7.9883x; 1.0830x over previous
import functools

import jax
import jax.numpy as jnp
from jax import lax
from jax.experimental import pallas as pl
from jax.experimental.pallas import tpu as pltpu
from jax.experimental.pallas import tpu_sc as plsc

_BATCH = 16384
_EMB = 32
_NC = 2
_NS = 16
_NW = _NC * _NS
_BPW = _BATCH // _NW
_CH = 32
_L = 16

_BT = 2048


def _sc_gather_one(idx, table, dummy):
    mesh = plsc.VectorSubcoreMesh(core_axis_name="c", subcore_axis_name="s")

    @functools.partial(
        pl.kernel,
        out_type=jax.ShapeDtypeStruct((_BATCH // 4, 128), jnp.float32),
        mesh=mesh,
        compiler_params=pltpu.CompilerParams(use_tc_tiling_on_sc=True,
                                             needs_layout_passes=False),
        scratch_types=[
            pltpu.VMEM((_BPW,), jnp.int32),
            pltpu.VMEM((_CH, 8, _EMB), jnp.float32),
            pltpu.VMEM((_CH, 8, _EMB), jnp.float32),
            pltpu.VMEM((_BPW // 4, 128), jnp.float32),
            pltpu.SemaphoreType.DMA,
            pltpu.SemaphoreType.DMA,
        ],
    )
    def gather(idx_hbm, tab_hbm, dummy_hbm, out_hbm,
               idx_v, slab_a, slab_b, out_v, sem_a, sem_b):
        wid = lax.axis_index("s") * _NC + lax.axis_index("c")
        base = pl.multiple_of(wid * _BPW, _BPW)
        base4 = pl.multiple_of(wid * (_BPW // 4), _BPW // 4)
        pltpu.sync_copy(idx_hbm.at[pl.ds(base, _BPW)], idx_v)

        def fire_chunk(c, slab_v, sem):
            def fire(g, carry):
                off = pl.multiple_of(c * _CH + g * _L, _L)
                vec = idx_v[pl.ds(off, _L)]
                for k2 in range(_L):
                    row = pl.multiple_of((vec[k2] // 8) * 8, 8)
                    pltpu.async_copy(tab_hbm.at[pl.ds(row, 8)],
                                     slab_v.at[g * _L + k2], sem)
                return carry

            lax.fori_loop(0, _CH // _L, fire, 0)

        def extract(slab_v, c):
            for g in range(_CH // _L):
                lrow = lax.iota(jnp.int32, _L) + g * _L
                off = pl.multiple_of(c * _CH + g * _L, _L)
                sub = lax.bitwise_and(idx_v[pl.ds(off, _L)], 7)
                grow = lrow + c * _CH
                drow = lax.shift_right_logical(grow, 2)
                dcol0 = lax.bitwise_and(grow, 3) * _EMB
                for j in range(_EMB):
                    vals = plsc.load_gather(
                        slab_v, [lrow, sub, jnp.full((_L,), j, jnp.int32)])
                    plsc.store_scatter(out_v, [drow, dcol0 + j], vals)

        nchunks = _BPW // _CH
        fire_chunk(0, slab_a, sem_a)

        def body(k, _):
            ca = 2 * k
            fire_chunk(ca + 1, slab_b, sem_b)
            pltpu.make_async_copy(dummy_hbm, slab_a, sem_a).wait()
            extract(slab_a, ca)

            @pl.when(k < nchunks // 2 - 1)
            def _():
                fire_chunk(ca + 2, slab_a, sem_a)

            pltpu.make_async_copy(dummy_hbm, slab_b, sem_b).wait()
            extract(slab_b, ca + 1)
            return 0

        lax.fori_loop(0, nchunks // 2, body, 0)

        pltpu.sync_copy(out_v, out_hbm.at[pl.ds(base4, _BPW // 4)])

    return gather(idx, table, dummy)


def _mlp_body(u_ref, i_ref, w1u_ref, w1i_ref, b1_ref, w2_ref, b2_ref,
              w3_ref, b3_ref, o_ref):
    dn = (((1,), (1,)), ((), ()))
    x1 = lax.dot_general(u_ref[...], w1u_ref[...], dn,
                         preferred_element_type=jnp.float32)
    x1 = x1 + lax.dot_general(i_ref[...], w1i_ref[...], dn,
                              preferred_element_type=jnp.float32)
    x1 = jnp.maximum(x1 + b1_ref[...], 0.0)
    x2 = lax.dot_general(x1, w2_ref[...], dn,
                         preferred_element_type=jnp.float32)
    x2 = jnp.maximum(x2 + b2_ref[...], 0.0)
    z = jnp.sum(x2 * w3_ref[...], axis=1, keepdims=True)
    z = z + b3_ref[0]
    o_ref[...] = 1.0 / (1.0 + jnp.exp(-z))


def _tc_mlp(u_emb, i_emb, W1u, W1i, b1r, W2, b2r, W3, b3):
    grid = (_BATCH // _BT,)
    full = lambda shape: pl.BlockSpec(shape, lambda i: (0, 0))
    return pl.pallas_call(
        _mlp_body,
        grid=grid,
        in_specs=[
            pl.BlockSpec((_BT, _EMB), lambda i: (i, 0)),
            pl.BlockSpec((_BT, _EMB), lambda i: (i, 0)),
            full(W1u.shape),
            full(W1i.shape),
            full(b1r.shape),
            full(W2.shape),
            full(b2r.shape),
            full(W3.shape),
            pl.BlockSpec(memory_space=pltpu.SMEM),
        ],
        out_specs=pl.BlockSpec((_BT, 1), lambda i: (i, 0)),
        out_shape=jax.ShapeDtypeStruct((_BATCH, 1), jnp.float32),
    )(u_emb, i_emb, W1u, W1i, b1r, W2, b2r, W3, b3)


def kernel(user_idx, item_idx, user_table, item_table, W1, b1, W2, b2, W3, b3):
    uidx = user_idx.astype(jnp.int32)
    iidx = item_idx.astype(jnp.int32)
    dummy = jnp.zeros((_CH, 8, _EMB), jnp.float32)
    u_out = _sc_gather_one(uidx, user_table, dummy)
    i_out = _sc_gather_one(iidx, item_table, dummy)
    u_emb = u_out.reshape(_BATCH, _EMB)
    i_emb = i_out.reshape(_BATCH, _EMB)
    W1u = W1[:, :_EMB]
    W1i = W1[:, _EMB:]
    return _tc_mlp(u_emb, i_emb, W1u, W1i,
                   b1.reshape(1, -1), W2, b2.reshape(1, -1),
                   W3, b3)
